# baseline (speedup 1.0000x reference)
import functools

import jax
import jax.numpy as jnp
from jax import lax
from jax.experimental import pallas as pl
from jax.experimental.pallas import tpu as pltpu
from jax.experimental.pallas import tpu_sc as plsc

NUM_AGENTS = 32768
NUM_ABS = 64
EMB_DIM = 256
ACT_DIM = 1024
BM = 2048

NC, NS, L = 2, 16, 16
NW = NC * NS
AG_PER_W = NUM_AGENTS // NW
CHUNK = 256

_P = (-2.786805564302052, 5.046852935538806, -3.492466042569591,
      1.593884548277405, -0.40486230941886187, 0.043428363332009945)
_LN2 = 0.6931471805599453
_INV_LN2 = 1.4426950408889634


def _neg_log2(x):
    i = plsc.bitcast(x, jnp.int32)
    e = (lax.shift_right_logical(i, 23) - 127).astype(jnp.float32)
    m = plsc.bitcast(
        lax.bitwise_or(lax.bitwise_and(i, 0x7FFFFF), 0x3F800000), jnp.float32)
    p = jnp.float32(_P[5])
    for k in (4, 3, 2, 1, 0):
        p = p * m + jnp.float32(_P[k])
    return -(e + p)


def _gumbel_score(u, l):
    d = jnp.float32(1.0) - u
    ser = d * (1.0 + d * (jnp.float32(1 / 2) + d * (jnp.float32(1 / 3)
              + d * (jnp.float32(1 / 4) + d * jnp.float32(1 / 5)))))
    y = jnp.where(u >= 0.875, ser * jnp.float32(_INV_LN2), _neg_log2(u))
    return l + jnp.float32(_LN2) * _neg_log2(y)


def _route_body(u_hbm, al_hbm, aa_hbm, out_hbm, u_v, al_v, aa_v, asg_v):
    wid = lax.axis_index("s") * NC + lax.axis_index("c")
    base = wid * AG_PER_W
    pltpu.sync_copy(aa_hbm, aa_v)

    def chunk_body(ci, _):
        cbase = base + ci * CHUNK
        pltpu.sync_copy(u_hbm.at[pl.ds(cbase * NUM_ABS, CHUNK * NUM_ABS)], u_v)
        pltpu.sync_copy(al_hbm.at[pl.ds(cbase * NUM_ABS, CHUNK * NUM_ABS)], al_v)

        def grp_body(gi, _):
            flat0 = (gi * L + lax.iota(jnp.int32, 16)) * NUM_ABS

            def j_body(j, carry):
                m, zi = carry
                flat = flat0 + j
                uv = plsc.load_gather(u_v, [flat])
                lv = plsc.load_gather(al_v, [flat])
                s = _gumbel_score(uv, lv)
                better = s > m
                return jnp.where(better, s, m), jnp.where(better, j, zi)

            m0 = jnp.full((16,), -jnp.inf, jnp.float32)
            z0 = jnp.zeros((16,), jnp.int32)
            _, zi = lax.fori_loop(0, NUM_ABS, j_body, (m0, z0))
            asg_v[pl.ds(gi * L, L)] = plsc.load_gather(aa_v, [zi])
            return 0

        lax.fori_loop(0, CHUNK // L, grp_body, 0)
        pltpu.sync_copy(asg_v, out_hbm.at[pl.ds(cbase, CHUNK)])
        return 0

    lax.fori_loop(0, AG_PER_W // CHUNK, chunk_body, 0)


@functools.partial(
    pl.kernel,
    out_type=jax.ShapeDtypeStruct((NUM_AGENTS,), jnp.float32),
    mesh=plsc.VectorSubcoreMesh(core_axis_name="c", subcore_axis_name="s"),
    scratch_types=[
        pltpu.VMEM((CHUNK * NUM_ABS,), jnp.float32),
        pltpu.VMEM((CHUNK * NUM_ABS,), jnp.float32),
        pltpu.VMEM((NUM_ABS,), jnp.float32),
        pltpu.VMEM((CHUNK,), jnp.float32),
    ],
    compiler_params=pltpu.CompilerParams(needs_layout_passes=False),
)
def _route_sc(u_hbm, al_hbm, aa_hbm, out_hbm, u_v, al_v, aa_v, asg_v):
    _route_body(u_hbm, al_hbm, aa_hbm, out_hbm, u_v, al_v, aa_v, asg_v)


def _dense_body(asg_ref, emb_ref, w1t_ref, w0_ref, b_ref, out_ref):
    acc = jnp.dot(emb_ref[...].astype(jnp.bfloat16),
                  w1t_ref[...].astype(jnp.bfloat16),
                  preferred_element_type=jnp.float32)
    logits = acc + asg_ref[...] * w0_ref[...] + b_ref[...]
    mx = jnp.max(logits, axis=1, keepdims=True)
    e = jnp.exp(logits - mx)
    out_ref[...] = e * (1.0 / jnp.sum(e, axis=1, keepdims=True))


@jax.jit
def kernel(abs_actions, gumbel_u, assigner_logits, emb_table, W, b):
    assigned = _route_sc(gumbel_u.reshape(-1), assigner_logits.reshape(-1),
                         abs_actions)
    w1t = W[:, 1:].T
    w0 = W[:, 0].reshape(1, ACT_DIM)
    br = b.reshape(1, ACT_DIM)
    grid = (NUM_AGENTS // BM,)
    return pl.pallas_call(
        _dense_body,
        grid=grid,
        in_specs=[
            pl.BlockSpec((BM, 1), lambda i: (i, 0)),
            pl.BlockSpec((BM, EMB_DIM), lambda i: (i, 0)),
            pl.BlockSpec((EMB_DIM, ACT_DIM), lambda i: (0, 0)),
            pl.BlockSpec((1, ACT_DIM), lambda i: (0, 0)),
            pl.BlockSpec((1, ACT_DIM), lambda i: (0, 0)),
        ],
        out_specs=pl.BlockSpec((BM, ACT_DIM), lambda i: (i, 0)),
        out_shape=jax.ShapeDtypeStruct((NUM_AGENTS, ACT_DIM), jnp.float32),
    )(assigned.reshape(NUM_AGENTS, 1), emb_table, w1t, w0, br)


# speedup vs baseline: 3.1893x; 3.1893x over previous
import jax
import jax.numpy as jnp
from jax.experimental import pallas as pl

NUM_AGENTS = 32768
NUM_ABS = 64
EMB_DIM = 256
ACT_DIM = 1024
BM = 2048


def _body(ut_ref, alt_ref, emb_ref, aa_ref, w1t_ref, w0_ref, b_ref, out_ref):
    s = alt_ref[...] - jnp.log(-jnp.log(ut_ref[...]))
    m = jnp.max(s, axis=0, keepdims=True)
    iota = jax.lax.broadcasted_iota(jnp.int32, s.shape, 0)
    idx = jnp.min(jnp.where(s >= m, iota, NUM_ABS), axis=0, keepdims=True)
    onehot_t = (iota == idx).astype(jnp.float32)
    assigned = jax.lax.dot_general(
        onehot_t, aa_ref[...], (((0,), (0,)), ((), ())),
        preferred_element_type=jnp.float32)

    acc = jnp.dot(emb_ref[...].astype(jnp.bfloat16),
                  w1t_ref[...].astype(jnp.bfloat16),
                  preferred_element_type=jnp.float32)
    logits = acc + assigned * w0_ref[...] + b_ref[...]

    mx = jnp.max(logits, axis=1, keepdims=True)
    e = jnp.exp(logits - mx)
    out_ref[...] = e * (1.0 / jnp.sum(e, axis=1, keepdims=True))


@jax.jit
def kernel(abs_actions, gumbel_u, assigner_logits, emb_table, W, b):
    ut = gumbel_u.T
    alt = assigner_logits.T
    w1t = W[:, 1:].T
    w0 = W[:, 0].reshape(1, ACT_DIM)
    br = b.reshape(1, ACT_DIM)
    aa = abs_actions.reshape(NUM_ABS, 1)

    grid = (NUM_AGENTS // BM,)
    return pl.pallas_call(
        _body,
        grid=grid,
        in_specs=[
            pl.BlockSpec((NUM_ABS, BM), lambda i: (0, i)),
            pl.BlockSpec((NUM_ABS, BM), lambda i: (0, i)),
            pl.BlockSpec((BM, EMB_DIM), lambda i: (i, 0)),
            pl.BlockSpec((NUM_ABS, 1), lambda i: (0, 0)),
            pl.BlockSpec((EMB_DIM, ACT_DIM), lambda i: (0, 0)),
            pl.BlockSpec((1, ACT_DIM), lambda i: (0, 0)),
            pl.BlockSpec((1, ACT_DIM), lambda i: (0, 0)),
        ],
        out_specs=pl.BlockSpec((BM, ACT_DIM), lambda i: (i, 0)),
        out_shape=jax.ShapeDtypeStruct((NUM_AGENTS, ACT_DIM), jnp.float32),
    )(ut, alt, emb_table, aa, w1t, w0, br)
